# baseline (speedup 1.0000x reference)
import functools

import jax
import jax.numpy as jnp
from jax import lax
from jax.experimental import pallas as pl
from jax.experimental.pallas import tpu as pltpu
from jax.experimental.pallas import tpu_sc as plsc

N = 10000
NP = 10240
E = 320000
EP = 327680
D = 128
NW = 32
EPW = EP // NW
CHUNK = 64
SLAB = 40
ROWS_PER_W = NP // 16

_mesh = plsc.VectorSubcoreMesh(core_axis_name="c", subcore_axis_name="s")


@functools.partial(
    pl.kernel,
    mesh=_mesh,
    out_type=[
        jax.ShapeDtypeStruct((EP // CHUNK, CHUNK), jnp.int32),
        jax.ShapeDtypeStruct((NP,), jnp.float32),
        jax.ShapeDtypeStruct((NP,), jnp.float32),
    ],
    scratch_types=[
        pltpu.VMEM_SHARED((NP,), jnp.float32),
        pltpu.VMEM((8, CHUNK), jnp.int32),
        pltpu.VMEM((8, CHUNK), jnp.int32),
        pltpu.VMEM((8, CHUNK), jnp.float32),
        pltpu.VMEM((8, CHUNK), jnp.int32),
        pltpu.VMEM((ROWS_PER_W,), jnp.float32),
    ],
)
def _edge_prep(src_h, dst_h, srcm_h, d0_h, d1_h, deg_sh, s2, d2, w2, m2, zb):
    c = lax.axis_index("c")
    s = lax.axis_index("s")
    w = c * 16 + s

    def _z(i, _):
        zb[pl.ds(i * 16, 16)] = jnp.zeros((16,), jnp.float32)
        return _

    lax.fori_loop(0, ROWS_PER_W // 16, _z, None)
    pltpu.sync_copy(zb, deg_sh.at[pl.ds(s * ROWS_PER_W, ROWS_PER_W)])
    plsc.subcore_barrier()

    iota16 = lax.iota(jnp.int32, 16)

    def _block(b, _):
        rowbase = w * (EPW // CHUNK) + b * 8
        pltpu.sync_copy(src_h.at[pl.ds(rowbase, 8)], s2)
        pltpu.sync_copy(dst_h.at[pl.ds(rowbase, 8)], d2)

        def _row(r, _):
            for k in range(CHUNK // 16):
                sl = pl.ds(k * 16, 16)
                sv = s2[r, sl]
                dv = d2[r, sl]
                keep = sv != dv
                w2[r, sl] = jnp.where(keep, 1.0, 0.0).astype(jnp.float32)
                m2[r, sl] = jnp.where(keep, sv, N + iota16)
            return _

        lax.fori_loop(0, 8, _row, None)
        for j in range(8):
            pltpu.sync_copy(w2.at[j], deg_sh.at[s2.at[j]], add=True)
        pltpu.sync_copy(m2, srcm_h.at[pl.ds(rowbase, 8)])
        return _

    lax.fori_loop(0, EPW // (8 * CHUNK), _block, None)
    plsc.subcore_barrier()

    sl = pl.ds(s * ROWS_PER_W, ROWS_PER_W)

    @pl.when(c == 0)
    def _():
        pltpu.sync_copy(deg_sh.at[sl], d0_h.at[sl])

    @pl.when(c == 1)
    def _():
        pltpu.sync_copy(deg_sh.at[sl], d1_h.at[sl])


@functools.partial(
    pl.kernel,
    mesh=_mesh,
    out_type=[
        jax.ShapeDtypeStruct((NP, D), jnp.float32),
        jax.ShapeDtypeStruct((NP, D), jnp.float32),
    ],
    scratch_types=[
        pltpu.VMEM_SHARED((NP, D), jnp.float32),
        pltpu.VMEM((SLAB, CHUNK), jnp.int32),
        pltpu.VMEM((SLAB, CHUNK), jnp.int32),
        pltpu.VMEM((CHUNK, D), jnp.float32),
        pltpu.VMEM((CHUNK, D), jnp.float32),
        pltpu.VMEM((CHUNK, D), jnp.float32),
        pltpu.VMEM((CHUNK, D), jnp.float32),
        pltpu.SemaphoreType.DMA,
        pltpu.SemaphoreType.DMA,
    ],
)
def _seg_sum(u_h, srcm_h, dst_h, sa_h, sb_h, acc_sh, s2, d2, r0, r1, r2, r3,
             gsem, ssem):
    c = lax.axis_index("c")
    s = lax.axis_index("s")
    w = c * 16 + s
    rows = (r0, r1, r2, r3)

    def _zrow(r, _):
        for k in range(D // 16):
            r0[r, pl.ds(k * 16, 16)] = jnp.zeros((16,), jnp.float32)
        return _

    lax.fori_loop(0, CHUNK, _zrow, None)
    for i in range(ROWS_PER_W // CHUNK):
        pltpu.sync_copy(r0, acc_sh.at[pl.ds(s * ROWS_PER_W + i * CHUNK, CHUNK)])
    plsc.subcore_barrier()

    def _drain_scatter():
        pltpu.make_async_copy(u_h.at[pl.ds(0, CHUNK)], r0, ssem).wait()

    for h in range(EPW // CHUNK // SLAB):
        base_ch = w * (EPW // CHUNK) + h * SLAB
        pltpu.sync_copy(srcm_h.at[pl.ds(base_ch, SLAB)], s2)
        pltpu.sync_copy(dst_h.at[pl.ds(base_ch, SLAB)], d2)

        gs = [pltpu.async_copy(u_h.at[s2.at[b]], rows[b], gsem)
              for b in range(4)]
        for b in range(4):
            gs[b].wait()
            pltpu.async_copy(rows[b], acc_sh.at[d2.at[b]], ssem, add=True)

        def _group(t, _):
            g2 = []
            for b in range(4):
                _drain_scatter()
                g2.append(pltpu.async_copy(u_h.at[s2.at[t * 4 + b]], rows[b],
                                           gsem))
            for b in range(4):
                g2[b].wait()
                pltpu.async_copy(rows[b], acc_sh.at[d2.at[t * 4 + b]], ssem,
                                 add=True)
            return _

        lax.fori_loop(1, SLAB // 4, _group, None)
        for _ in range(4):
            _drain_scatter()
    plsc.subcore_barrier()

    sl = pl.ds(s * ROWS_PER_W, ROWS_PER_W)

    @pl.when(c == 0)
    def _():
        pltpu.sync_copy(acc_sh.at[sl], sa_h.at[sl])

    @pl.when(c == 1)
    def _():
        pltpu.sync_copy(acc_sh.at[sl], sb_h.at[sl])


R = 1024
_grid = (NP // R,)
_rowspec = pl.BlockSpec((R, D), lambda i: (i, 0))
_colspec = pl.BlockSpec((R, 1), lambda i: (i, 0))
_wspec = pl.BlockSpec((D, D), lambda i: (0, 0))
_bspec = pl.BlockSpec((1, D), lambda i: (0, 0))


def _silu(h):
    return h * (1.0 / (1.0 + jnp.exp(-h)))


def _dis(d0, d1):
    deg = d0 + d1
    return jnp.where(deg > 0, lax.rsqrt(jnp.where(deg > 0, deg, 1.0)), 0.0)


def _tc_in_body(x, d0, d1, Wi, bi, W1, W0, u_o, v_o):
    dis = _dis(d0[...], d1[...])
    h = _silu(jnp.dot(x[...], Wi[...], preferred_element_type=jnp.float32)
              + bi[...])
    u_o[...] = jnp.dot(dis * h, W1[...], preferred_element_type=jnp.float32)
    v_o[...] = jnp.dot(h, W0[...], preferred_element_type=jnp.float32)


def _tc_mid_body(v, sa, sb, d0, d1, b, W1, W0, u_o, v_o):
    dis = _dis(d0[...], d1[...])
    h = _silu(v[...] - dis * (sa[...] + sb[...]) + b[...])
    u_o[...] = jnp.dot(dis * h, W1[...], preferred_element_type=jnp.float32)
    v_o[...] = jnp.dot(h, W0[...], preferred_element_type=jnp.float32)


def _tc_out_body(v, sa, sb, d0, d1, b, Wo, bo, out_o):
    dis = _dis(d0[...], d1[...])
    h = _silu(v[...] - dis * (sa[...] + sb[...]) + b[...])
    out_o[...] = jnp.dot(h, Wo[...], preferred_element_type=jnp.float32) + bo[...]


_tc_in = pl.pallas_call(
    _tc_in_body,
    grid=_grid,
    in_specs=[_rowspec, _colspec, _colspec, _wspec, _bspec, _wspec, _wspec],
    out_specs=[_rowspec, _rowspec],
    out_shape=[jax.ShapeDtypeStruct((NP, D), jnp.float32)] * 2,
)

_tc_mid = pl.pallas_call(
    _tc_mid_body,
    grid=_grid,
    in_specs=[_rowspec, _rowspec, _rowspec, _colspec, _colspec, _bspec,
              _wspec, _wspec],
    out_specs=[_rowspec, _rowspec],
    out_shape=[jax.ShapeDtypeStruct((NP, D), jnp.float32)] * 2,
)

_tc_out = pl.pallas_call(
    _tc_out_body,
    grid=_grid,
    in_specs=[_rowspec, _rowspec, _rowspec, _colspec, _colspec, _bspec,
              _wspec, _bspec],
    out_specs=_rowspec,
    out_shape=jax.ShapeDtypeStruct((NP, D), jnp.float32),
)


def kernel(x, edge_index, W_in, b_in, conv0_W0, conv0_W1, conv0_b,
           conv1_W0, conv1_W1, conv1_b, W_out, b_out):
    src = edge_index[0]
    dst = edge_index[1]
    pad = (jnp.arange(EP - E, dtype=jnp.int32) * 37) % N
    src2 = jnp.concatenate([src, pad]).reshape(EP // CHUNK, CHUNK)
    dst2 = jnp.concatenate([dst, pad]).reshape(EP // CHUNK, CHUNK)

    srcm2, d0, d1 = _edge_prep(src2, dst2)
    d0c = d0.reshape(NP, 1)
    d1c = d1.reshape(NP, 1)

    xp = jnp.pad(x, ((0, NP - N), (0, 0)))
    bi = b_in.reshape(1, D)
    b0 = conv0_b.reshape(1, D)
    b1 = conv1_b.reshape(1, D)
    bo = b_out.reshape(1, D)

    u0, v0 = _tc_in(xp, d0c, d1c, W_in, bi, conv0_W1, conv0_W0)
    sa0, sb0 = _seg_sum(u0, srcm2, dst2)
    u1, v1 = _tc_mid(v0, sa0, sb0, d0c, d1c, b0, conv1_W1, conv1_W0)
    sa1, sb1 = _seg_sum(u1, srcm2, dst2)
    out = _tc_out(v1, sa1, sb1, d0c, d1c, b1, W_out, bo)
    return out[:N]


# speedup vs baseline: 1.0899x; 1.0899x over previous
import functools

import jax
import jax.numpy as jnp
from jax import lax
from jax.experimental import pallas as pl
from jax.experimental.pallas import tpu as pltpu
from jax.experimental.pallas import tpu_sc as plsc

N = 10000
NP = 10240
E = 320000
EP = 327680
D = 128
NW = 32
EPW = EP // NW
CHUNK = 128
SLAB = 40
ROWS_PER_W = NP // 16

_mesh = plsc.VectorSubcoreMesh(core_axis_name="c", subcore_axis_name="s")


@functools.partial(
    pl.kernel,
    mesh=_mesh,
    out_type=[
        jax.ShapeDtypeStruct((EP // CHUNK, CHUNK), jnp.int32),
        jax.ShapeDtypeStruct((NP,), jnp.float32),
        jax.ShapeDtypeStruct((NP,), jnp.float32),
    ],
    scratch_types=[
        pltpu.VMEM_SHARED((NP,), jnp.float32),
        pltpu.VMEM((8, CHUNK), jnp.int32),
        pltpu.VMEM((8, CHUNK), jnp.int32),
        pltpu.VMEM((8, CHUNK), jnp.float32),
        pltpu.VMEM((8, CHUNK), jnp.int32),
        pltpu.VMEM((ROWS_PER_W,), jnp.float32),
    ],
)
def _edge_prep(src_h, dst_h, srcm_h, d0_h, d1_h, deg_sh, s2, d2, w2, m2, zb):
    c = lax.axis_index("c")
    s = lax.axis_index("s")
    w = c * 16 + s

    def _z(i, _):
        zb[pl.ds(i * 16, 16)] = jnp.zeros((16,), jnp.float32)
        return _

    lax.fori_loop(0, ROWS_PER_W // 16, _z, None)
    pltpu.sync_copy(zb, deg_sh.at[pl.ds(s * ROWS_PER_W, ROWS_PER_W)])
    plsc.subcore_barrier()

    iota16 = lax.iota(jnp.int32, 16)

    def _block(b, _):
        rowbase = w * (EPW // CHUNK) + b * 8
        pltpu.sync_copy(src_h.at[pl.ds(rowbase, 8)], s2)
        pltpu.sync_copy(dst_h.at[pl.ds(rowbase, 8)], d2)

        def _row(r, _):
            for k in range(CHUNK // 16):
                sl = pl.ds(k * 16, 16)
                sv = s2[r, sl]
                dv = d2[r, sl]
                keep = sv != dv
                w2[r, sl] = jnp.where(keep, 1.0, 0.0).astype(jnp.float32)
                m2[r, sl] = jnp.where(keep, sv, N + iota16)
            return _

        lax.fori_loop(0, 8, _row, None)
        for j in range(8):
            pltpu.sync_copy(w2.at[j], deg_sh.at[s2.at[j]], add=True)
        pltpu.sync_copy(m2, srcm_h.at[pl.ds(rowbase, 8)])
        return _

    lax.fori_loop(0, EPW // (8 * CHUNK), _block, None)
    plsc.subcore_barrier()

    sl = pl.ds(s * ROWS_PER_W, ROWS_PER_W)

    @pl.when(c == 0)
    def _():
        pltpu.sync_copy(deg_sh.at[sl], d0_h.at[sl])

    @pl.when(c == 1)
    def _():
        pltpu.sync_copy(deg_sh.at[sl], d1_h.at[sl])


@functools.partial(
    pl.kernel,
    mesh=_mesh,
    out_type=[
        jax.ShapeDtypeStruct((NP, D), jnp.float32),
        jax.ShapeDtypeStruct((NP, D), jnp.float32),
    ],
    scratch_types=[
        pltpu.VMEM_SHARED((NP, D), jnp.float32),
        pltpu.VMEM((SLAB, CHUNK), jnp.int32),
        pltpu.VMEM((SLAB, CHUNK), jnp.int32),
        pltpu.VMEM((CHUNK, D), jnp.float32),
        pltpu.VMEM((CHUNK, D), jnp.float32),
        pltpu.SemaphoreType.DMA,
        pltpu.SemaphoreType.DMA,
    ],
)
def _seg_sum(u_h, srcm_h, dst_h, sa_h, sb_h, acc_sh, s2, d2, r0, r1,
             gsem, ssem):
    c = lax.axis_index("c")
    s = lax.axis_index("s")
    w = c * 16 + s
    rows = (r0, r1)

    def _zrow(r, _):
        for k in range(D // 16):
            r0[r, pl.ds(k * 16, 16)] = jnp.zeros((16,), jnp.float32)
        return _

    lax.fori_loop(0, CHUNK, _zrow, None)
    for i in range(ROWS_PER_W // CHUNK):
        pltpu.sync_copy(r0, acc_sh.at[pl.ds(s * ROWS_PER_W + i * CHUNK, CHUNK)])
    plsc.subcore_barrier()

    def _drain_scatter():
        pltpu.make_async_copy(u_h.at[pl.ds(0, CHUNK)], r0, ssem).wait()

    for h in range(EPW // CHUNK // SLAB):
        base_ch = w * (EPW // CHUNK) + h * SLAB
        pltpu.sync_copy(srcm_h.at[pl.ds(base_ch, SLAB)], s2)
        pltpu.sync_copy(dst_h.at[pl.ds(base_ch, SLAB)], d2)

        gs = [pltpu.async_copy(u_h.at[s2.at[b]], rows[b], gsem)
              for b in range(2)]
        for b in range(2):
            gs[b].wait()
            pltpu.async_copy(rows[b], acc_sh.at[d2.at[b]], ssem, add=True)

        def _group(t, _):
            g2 = []
            for b in range(2):
                _drain_scatter()
                g2.append(pltpu.async_copy(u_h.at[s2.at[t * 2 + b]], rows[b],
                                           gsem))
            for b in range(2):
                g2[b].wait()
                pltpu.async_copy(rows[b], acc_sh.at[d2.at[t * 2 + b]], ssem,
                                 add=True)
            return _

        lax.fori_loop(1, SLAB // 2, _group, None)
        for _ in range(2):
            _drain_scatter()
    plsc.subcore_barrier()

    sl = pl.ds(s * ROWS_PER_W, ROWS_PER_W)

    @pl.when(c == 0)
    def _():
        pltpu.sync_copy(acc_sh.at[sl], sa_h.at[sl])

    @pl.when(c == 1)
    def _():
        pltpu.sync_copy(acc_sh.at[sl], sb_h.at[sl])


R = 1024
_grid = (NP // R,)
_rowspec = pl.BlockSpec((R, D), lambda i: (i, 0))
_colspec = pl.BlockSpec((R, 1), lambda i: (i, 0))
_wspec = pl.BlockSpec((D, D), lambda i: (0, 0))
_bspec = pl.BlockSpec((1, D), lambda i: (0, 0))


def _silu(h):
    return h * (1.0 / (1.0 + jnp.exp(-h)))


def _dis(d0, d1):
    deg = d0 + d1
    return jnp.where(deg > 0, lax.rsqrt(jnp.where(deg > 0, deg, 1.0)), 0.0)


def _tc_h_body(x, Wi, bi, h_o):
    h_o[...] = _silu(jnp.dot(x[...], Wi[...],
                             preferred_element_type=jnp.float32) + bi[...])


def _tc_in_body(h, d0, d1, W1, W0, u_o, v_o):
    dis = _dis(d0[...], d1[...])
    h = h[...]
    u_o[...] = jnp.dot(dis * h, W1[...], preferred_element_type=jnp.float32)
    v_o[...] = jnp.dot(h, W0[...], preferred_element_type=jnp.float32)


def _tc_mid_body(v, sa, sb, d0, d1, b, W1, W0, u_o, v_o):
    dis = _dis(d0[...], d1[...])
    h = _silu(v[...] - dis * (sa[...] + sb[...]) + b[...])
    u_o[...] = jnp.dot(dis * h, W1[...], preferred_element_type=jnp.float32)
    v_o[...] = jnp.dot(h, W0[...], preferred_element_type=jnp.float32)


def _tc_out_body(v, sa, sb, d0, d1, b, Wo, bo, out_o):
    dis = _dis(d0[...], d1[...])
    h = _silu(v[...] - dis * (sa[...] + sb[...]) + b[...])
    out_o[...] = jnp.dot(h, Wo[...], preferred_element_type=jnp.float32) + bo[...]


_tc_h = pl.pallas_call(
    _tc_h_body,
    grid=_grid,
    in_specs=[_rowspec, _wspec, _bspec],
    out_specs=_rowspec,
    out_shape=jax.ShapeDtypeStruct((NP, D), jnp.float32),
)

_tc_in = pl.pallas_call(
    _tc_in_body,
    grid=_grid,
    in_specs=[_rowspec, _colspec, _colspec, _wspec, _wspec],
    out_specs=[_rowspec, _rowspec],
    out_shape=[jax.ShapeDtypeStruct((NP, D), jnp.float32)] * 2,
)

_tc_mid = pl.pallas_call(
    _tc_mid_body,
    grid=_grid,
    in_specs=[_rowspec, _rowspec, _rowspec, _colspec, _colspec, _bspec,
              _wspec, _wspec],
    out_specs=[_rowspec, _rowspec],
    out_shape=[jax.ShapeDtypeStruct((NP, D), jnp.float32)] * 2,
)

_tc_out = pl.pallas_call(
    _tc_out_body,
    grid=_grid,
    in_specs=[_rowspec, _rowspec, _rowspec, _colspec, _colspec, _bspec,
              _wspec, _bspec],
    out_specs=_rowspec,
    out_shape=jax.ShapeDtypeStruct((NP, D), jnp.float32),
)


def kernel(x, edge_index, W_in, b_in, conv0_W0, conv0_W1, conv0_b,
           conv1_W0, conv1_W1, conv1_b, W_out, b_out):
    src = edge_index[0]
    dst = edge_index[1]
    pad = (jnp.arange(EP - E, dtype=jnp.int32) * 37) % N
    src2 = jnp.concatenate([src, pad]).reshape(EP // CHUNK, CHUNK)
    dst2 = jnp.concatenate([dst, pad]).reshape(EP // CHUNK, CHUNK)

    srcm2, d0, d1 = _edge_prep(src2, dst2)
    d0c = d0.reshape(NP, 1)
    d1c = d1.reshape(NP, 1)

    xp = jnp.pad(x, ((0, NP - N), (0, 0)))
    bi = b_in.reshape(1, D)
    b0 = conv0_b.reshape(1, D)
    b1 = conv1_b.reshape(1, D)
    bo = b_out.reshape(1, D)

    h0 = _tc_h(xp, W_in, bi)
    u0, v0 = _tc_in(h0, d0c, d1c, conv0_W1, conv0_W0)
    sa0, sb0 = _seg_sum(u0, srcm2, dst2)
    u1, v1 = _tc_mid(v0, sa0, sb0, d0c, d1c, b0, conv1_W1, conv1_W0)
    sa1, sb1 = _seg_sum(u1, srcm2, dst2)
    out = _tc_out(v1, sa1, sb1, d0c, d1c, b1, W_out, bo)
    return out[:N]
